# baseline (speedup 1.0000x reference)
import functools

import jax
import jax.numpy as jnp
from jax import lax
from jax.experimental import pallas as pl
from jax.experimental.pallas import tpu as pltpu
from jax.experimental.pallas import tpu_sc as plsc

RES = 64
V = RES ** 3
EPS = 1e-08
B = 4
C = 32
N = 16384

NW = 32
PTS_PER_W = (B * N) // NW
CHUNK = 128
NCHUNK = PTS_PER_W // CHUNK
LANES = 16


def _prep_body(pts_ref, mpe_ref, idx_ref, w_ref):
    pts = pts_ref[...]
    pmin = jnp.min(pts, axis=2, keepdims=True)
    pn = pts - pmin
    pn = pn / mpe_ref[0, 0]
    vox = pn * (RES - 1.0)
    il = jnp.floor(vox).astype(jnp.int32)
    ir = jnp.ceil(vox).astype(jnp.int32)
    wl = 1.0 - vox
    wr = vox
    boff = (jnp.arange(B, dtype=jnp.int32) * V)[:, None]
    for k in range(8):
        bx, by, bz = k & 1, (k >> 1) & 1, (k >> 2) & 1
        ix = (ir if bx else il)[:, 0]
        iy = (ir if by else il)[:, 1]
        iz = (ir if bz else il)[:, 2]
        m = iy * RES + iz
        idxk = (
            boff + ix * (RES * RES)
            + 4 * (m & (1024 - 1)) + (m >> 10)
        )
        for b in range(B):
            idx_ref[pl.ds(k, 1), pl.ds(b * N, N)] = idxk[b:b + 1]
    for a in range(3):
        for b in range(B):
            w_ref[pl.ds(a, 1), pl.ds(b * N, N)] = wl[b:b + 1, a]
            w_ref[pl.ds(3 + a, 1), pl.ds(b * N, N)] = wr[b:b + 1, a]


def _prep(pts, mpe):
    return pl.pallas_call(
        _prep_body,
        out_shape=[
            jax.ShapeDtypeStruct((8, B * N), jnp.int32),
            jax.ShapeDtypeStruct((6, B * N), jnp.float32),
        ],
    )(pts, mpe)


def _tx_body(f_ref, o_ref):
    val = f_ref[0, :, 0]
    v2 = val.reshape(C, RES * RES)
    t128 = jnp.concatenate(
        [v2[:, 1024 * q:1024 * (q + 1)] for q in range(4)], axis=0
    )
    o_ref[...] = jnp.transpose(t128, (1, 0))


def _tx(feat):
    return pl.pallas_call(
        _tx_body,
        grid=(B, RES),
        in_specs=[
            pl.BlockSpec(
                (1, C, 1, RES, RES), lambda b, x: (b, 0, x, 0, 0)
            )
        ],
        out_specs=pl.BlockSpec(
            (RES * RES // 4, 4 * C), lambda b, x: (b * RES + x, 0)
        ),
        out_shape=jax.ShapeDtypeStruct((B * V // 4, 4 * C), jnp.float32),
    )(feat)


def _sc_body(
    table, idx_hbm, w_hbm, out_hbm,
    idx_all, w0, w1, w2, w3, w4, w5, rows_a, rows_b,
    out_flat, out_a, out_b, semr, semo_a, semo_b,
):
    wid = lax.axis_index("s") * 2 + lax.axis_index("c")
    qbase0 = wid * PTS_PER_W
    bb = wid // (N // PTS_PER_W)
    col0 = qbase0 - bb * N
    wrefs = (w0, w1, w2, w3, w4, w5)
    iot = lax.iota(jnp.int32, LANES)

    pltpu.sync_copy(idx_hbm.at[:, pl.ds(qbase0, PTS_PER_W)], idx_all)
    for a in range(6):
        pltpu.sync_copy(w_hbm.at[a, pl.ds(qbase0, PTS_PER_W)], wrefs[a])

    def fire(g, rows):
        return [
            pltpu.async_copy(
                table.at[idx_all.at[k, pl.ds(g * CHUNK, CHUNK)]],
                rows.at[pl.ds(k * CHUNK, CHUNK)],
                semr,
            )
            for k in range(8)
        ]

    def compute(g, rows, outb):
        def p_body(p, carry_p):
            pv = jnp.full((LANES,), g * CHUNK, jnp.int32) + p
            wlx = plsc.load_gather(w0, [pv])
            wly = plsc.load_gather(w1, [pv])
            wlz = plsc.load_gather(w2, [pv])
            wrx = plsc.load_gather(w3, [pv])
            wry = plsc.load_gather(w4, [pv])
            wrz = plsc.load_gather(w5, [pv])
            for h in range(2):
                f = [
                    rows[k * CHUNK + p, pl.ds(h * LANES, LANES)]
                    for k in range(8)
                ]
                f00 = f[0] * wlx + f[1] * wrx
                f10 = f[2] * wlx + f[3] * wrx
                f01 = f[4] * wlx + f[5] * wrx
                f11 = f[6] * wlx + f[7] * wrx
                g0 = f00 * wly + f10 * wry
                g1 = f01 * wly + f11 * wry
                out_flat[pl.ds(p * C + h * LANES, LANES)] = (
                    g0 * wlz + g1 * wrz
                )
            return carry_p

        lax.fori_loop(0, CHUNK, p_body, 0)

        def c_body(c, carry_c):
            for j in range(CHUNK // LANES):
                vals = plsc.load_gather(
                    out_flat, [(iot + j * LANES) * C + c]
                )
                outb[c, pl.ds(j * LANES, LANES)] = vals
            return carry_c

        lax.fori_loop(0, C, c_body, 0)

    def out_copy(g, outb, semo):
        return pltpu.make_async_copy(
            outb,
            out_hbm.at[pl.ds(bb * C, C), pl.ds(col0 + g * CHUNK, CHUNK)],
            semo,
        )

    row_cp = {0: fire(0, rows_a), 1: fire(1, rows_b)}
    out_cp = {}
    for g in range(NCHUNK):
        rows = rows_a if g % 2 == 0 else rows_b
        outb = out_a if g % 2 == 0 else out_b
        semo = semo_a if g % 2 == 0 else semo_b
        for cp in row_cp.pop(g):
            cp.wait()
        if g >= 2:
            out_cp.pop(g - 2).wait()
        compute(g, rows, outb)
        cp = out_copy(g, outb, semo)
        cp.start()
        out_cp[g] = cp
        if g + 2 < NCHUNK:
            row_cp[g + 2] = fire(g + 2, rows)
    out_cp.pop(NCHUNK - 2).wait()
    out_cp.pop(NCHUNK - 1).wait()


@functools.lru_cache(maxsize=1)
def _make_sc_gather():
    mesh = plsc.VectorSubcoreMesh(core_axis_name="c", subcore_axis_name="s")
    return pl.kernel(
        _sc_body,
        out_type=jax.ShapeDtypeStruct((B * C, N), jnp.float32),
        mesh=mesh,
        compiler_params=pltpu.CompilerParams(
            use_tc_tiling_on_sc=False, needs_layout_passes=False
        ),
        scratch_types=[
            pltpu.VMEM((8, PTS_PER_W), jnp.int32),
            pltpu.VMEM((PTS_PER_W,), jnp.float32),
            pltpu.VMEM((PTS_PER_W,), jnp.float32),
            pltpu.VMEM((PTS_PER_W,), jnp.float32),
            pltpu.VMEM((PTS_PER_W,), jnp.float32),
            pltpu.VMEM((PTS_PER_W,), jnp.float32),
            pltpu.VMEM((PTS_PER_W,), jnp.float32),
            pltpu.VMEM((8 * CHUNK, C), jnp.float32),
            pltpu.VMEM((8 * CHUNK, C), jnp.float32),
            pltpu.VMEM((CHUNK * C,), jnp.float32),
            pltpu.VMEM((C, CHUNK), jnp.float32),
            pltpu.VMEM((C, CHUNK), jnp.float32),
            pltpu.SemaphoreType.DMA,
            pltpu.SemaphoreType.DMA,
            pltpu.SemaphoreType.DMA,
        ],
    )


def kernel(pts, feat):
    pmin = jnp.min(pts, axis=2, keepdims=True)
    pn0 = pts - pmin
    mpe = jnp.max(jnp.sqrt(jnp.sum(pn0 * pn0, axis=1))) + EPS
    idx8, w8 = _prep(pts, mpe.reshape(1, 1))
    table = _tx(feat).reshape(B * V, C)
    out = _make_sc_gather()(table, idx8, w8)
    return out.reshape(B, C, N)


# speedup vs baseline: 1.3024x; 1.3024x over previous
import functools

import jax
import jax.numpy as jnp
from jax import lax
from jax.experimental import pallas as pl
from jax.experimental.pallas import tpu as pltpu
from jax.experimental.pallas import tpu_sc as plsc

RES = 64
V = RES ** 3
EPS = 1e-08
B = 4
C = 32
N = 16384

NW = 32
PTS_PER_W = (B * N) // NW
CHUNK = 128
NCHUNK = PTS_PER_W // CHUNK
LANES = 16


def _prep_body(pts_ref, mpe_ref, idx_ref, w_ref):
    pts = pts_ref[...]
    pmin = jnp.min(pts, axis=2, keepdims=True)
    pn = pts - pmin
    pn = pn / mpe_ref[0, 0]
    vox = pn * (RES - 1.0)
    il = jnp.floor(vox).astype(jnp.int32)
    ir = jnp.ceil(vox).astype(jnp.int32)
    wl = 1.0 - vox
    wr = vox
    boff = (jnp.arange(B, dtype=jnp.int32) * V)[:, None]
    for k in range(8):
        bx, by, bz = k & 1, (k >> 1) & 1, (k >> 2) & 1
        ix = (ir if bx else il)[:, 0]
        iy = (ir if by else il)[:, 1]
        iz = (ir if bz else il)[:, 2]
        m = iy * RES + iz
        idxk = (
            boff + ix * (RES * RES)
            + 4 * (m & (1024 - 1)) + (m >> 10)
        )
        for b in range(B):
            idx_ref[pl.ds(k, 1), pl.ds(b * N, N)] = idxk[b:b + 1]
    for a in range(3):
        for b in range(B):
            w_ref[pl.ds(a, 1), pl.ds(b * N, N)] = wl[b:b + 1, a]
            w_ref[pl.ds(3 + a, 1), pl.ds(b * N, N)] = wr[b:b + 1, a]


def _prep(pts, mpe):
    return pl.pallas_call(
        _prep_body,
        out_shape=[
            jax.ShapeDtypeStruct((8, B * N), jnp.int32),
            jax.ShapeDtypeStruct((6, B * N), jnp.float32),
        ],
    )(pts, mpe)


XS = 2


def _tx_body(f_ref, o_ref):
    for s in range(XS):
        val = f_ref[0, :, s]
        v2 = val.reshape(C, RES * RES)
        t128 = jnp.concatenate(
            [v2[:, 1024 * q:1024 * (q + 1)] for q in range(4)], axis=0
        )
        o_ref[pl.ds(s * 1024, 1024), :] = jnp.transpose(t128, (1, 0))


def _tx(feat):
    return pl.pallas_call(
        _tx_body,
        grid=(B, RES // XS),
        in_specs=[
            pl.BlockSpec(
                (1, C, XS, RES, RES), lambda b, x: (b, 0, x, 0, 0)
            )
        ],
        out_specs=pl.BlockSpec(
            (XS * RES * RES // 4, 4 * C),
            lambda b, x: (b * (RES // XS) + x, 0),
        ),
        out_shape=jax.ShapeDtypeStruct((B * V // 4, 4 * C), jnp.float32),
    )(feat)


def _sc_body(
    table, idx_hbm, w_hbm, out_hbm,
    idx_all, w0, w1, w2, w3, w4, w5, rows_a, rows_b,
    out_a, out_b, semr, semo_a, semo_b,
):
    wid = lax.axis_index("s") * 2 + lax.axis_index("c")
    qbase0 = wid * PTS_PER_W
    wrefs = (w0, w1, w2, w3, w4, w5)

    pltpu.sync_copy(idx_hbm.at[:, pl.ds(qbase0, PTS_PER_W)], idx_all)
    for a in range(6):
        pltpu.sync_copy(w_hbm.at[a, pl.ds(qbase0, PTS_PER_W)], wrefs[a])

    def fire(g, rows):
        return [
            pltpu.async_copy(
                table.at[idx_all.at[k, pl.ds(g * CHUNK, CHUNK)]],
                rows.at[pl.ds(k * CHUNK, CHUNK)],
                semr,
            )
            for k in range(8)
        ]

    def compute(g, rows, outb):
        def p_body(p, carry_p):
            pv = jnp.full((LANES,), g * CHUNK, jnp.int32) + p
            wlx = plsc.load_gather(w0, [pv])
            wly = plsc.load_gather(w1, [pv])
            wlz = plsc.load_gather(w2, [pv])
            wrx = plsc.load_gather(w3, [pv])
            wry = plsc.load_gather(w4, [pv])
            wrz = plsc.load_gather(w5, [pv])
            for h in range(2):
                f = [
                    rows[k * CHUNK + p, pl.ds(h * LANES, LANES)]
                    for k in range(8)
                ]
                f00 = f[0] * wlx + f[1] * wrx
                f10 = f[2] * wlx + f[3] * wrx
                f01 = f[4] * wlx + f[5] * wrx
                f11 = f[6] * wlx + f[7] * wrx
                g0 = f00 * wly + f10 * wry
                g1 = f01 * wly + f11 * wry
                outb[p, pl.ds(h * LANES, LANES)] = g0 * wlz + g1 * wrz
            return carry_p

        lax.fori_loop(0, CHUNK, p_body, 0)

    def out_copy(g, outb, semo):
        return pltpu.make_async_copy(
            outb, out_hbm.at[pl.ds(qbase0 + g * CHUNK, CHUNK)], semo
        )

    row_cp = {0: fire(0, rows_a), 1: fire(1, rows_b)}
    out_cp = {}
    for g in range(NCHUNK):
        rows = rows_a if g % 2 == 0 else rows_b
        outb = out_a if g % 2 == 0 else out_b
        semo = semo_a if g % 2 == 0 else semo_b
        for cp in row_cp.pop(g):
            cp.wait()
        if g >= 2:
            out_cp.pop(g - 2).wait()
        compute(g, rows, outb)
        cp = out_copy(g, outb, semo)
        cp.start()
        out_cp[g] = cp
        if g + 2 < NCHUNK:
            row_cp[g + 2] = fire(g + 2, rows)
    out_cp.pop(NCHUNK - 2).wait()
    out_cp.pop(NCHUNK - 1).wait()


@functools.lru_cache(maxsize=1)
def _make_sc_gather():
    mesh = plsc.VectorSubcoreMesh(core_axis_name="c", subcore_axis_name="s")
    return pl.kernel(
        _sc_body,
        out_type=jax.ShapeDtypeStruct((B * N, C), jnp.float32),
        mesh=mesh,
        compiler_params=pltpu.CompilerParams(
            use_tc_tiling_on_sc=False, needs_layout_passes=False
        ),
        scratch_types=[
            pltpu.VMEM((8, PTS_PER_W), jnp.int32),
            pltpu.VMEM((PTS_PER_W,), jnp.float32),
            pltpu.VMEM((PTS_PER_W,), jnp.float32),
            pltpu.VMEM((PTS_PER_W,), jnp.float32),
            pltpu.VMEM((PTS_PER_W,), jnp.float32),
            pltpu.VMEM((PTS_PER_W,), jnp.float32),
            pltpu.VMEM((PTS_PER_W,), jnp.float32),
            pltpu.VMEM((8 * CHUNK, C), jnp.float32),
            pltpu.VMEM((8 * CHUNK, C), jnp.float32),
            pltpu.VMEM((CHUNK, C), jnp.float32),
            pltpu.VMEM((CHUNK, C), jnp.float32),
            pltpu.SemaphoreType.DMA,
            pltpu.SemaphoreType.DMA,
            pltpu.SemaphoreType.DMA,
        ],
    )


def kernel(pts, feat):
    pmin = jnp.min(pts, axis=2, keepdims=True)
    pn0 = pts - pmin
    mpe = jnp.max(jnp.sqrt(jnp.sum(pn0 * pn0, axis=1))) + EPS
    idx8, w8 = _prep(pts, mpe.reshape(1, 1))
    table = _tx(feat).reshape(B * V, C)
    out = _make_sc_gather()(table, idx8, w8)
    return out.reshape(B, N, C).transpose(0, 2, 1)


# speedup vs baseline: 1.5015x; 1.1529x over previous
import functools

import jax
import jax.numpy as jnp
from jax import lax
from jax.experimental import pallas as pl
from jax.experimental.pallas import tpu as pltpu
from jax.experimental.pallas import tpu_sc as plsc

RES = 64
V = RES ** 3
EPS = 1e-08
B = 4
C = 32
N = 16384

NW = 32
PTS_PER_W = (B * N) // NW
CHUNK = 128
NCHUNK = PTS_PER_W // CHUNK
LANES = 16


def _prep_body(pts_ref, mpe_ref, idx_ref, w_ref):
    pts = pts_ref[...]
    pmin = jnp.min(pts, axis=2, keepdims=True)
    pn = pts - pmin
    pn = pn / mpe_ref[0, 0]
    vox = pn * (RES - 1.0)
    il = jnp.floor(vox).astype(jnp.int32)
    ir = jnp.ceil(vox).astype(jnp.int32)
    wl = 1.0 - vox
    wr = vox
    boff = (jnp.arange(B, dtype=jnp.int32) * V)[:, None]
    for k in range(8):
        bx, by, bz = k & 1, (k >> 1) & 1, (k >> 2) & 1
        ix = (ir if bx else il)[:, 0]
        iy = (ir if by else il)[:, 1]
        iz = (ir if bz else il)[:, 2]
        m = iy * RES + iz
        idxk = (
            boff + ix * (RES * RES)
            + 4 * (m & (1024 - 1)) + (m >> 10)
        )
        for b in range(B):
            idx_ref[pl.ds(k, 1), pl.ds(b * N, N)] = idxk[b:b + 1]
    for a in range(3):
        for b in range(B):
            w_ref[pl.ds(a, 1), pl.ds(b * N, N)] = wl[b:b + 1, a]
            w_ref[pl.ds(3 + a, 1), pl.ds(b * N, N)] = wr[b:b + 1, a]


def _prep(pts, mpe):
    return pl.pallas_call(
        _prep_body,
        out_shape=[
            jax.ShapeDtypeStruct((8, B * N), jnp.int32),
            jax.ShapeDtypeStruct((6, B * N), jnp.float32),
        ],
    )(pts, mpe)


XS = 4


def _tx_body(f_ref, o_ref):
    for s in range(XS):
        val = f_ref[0, :, s]
        v2 = val.reshape(C, RES * RES)
        t128 = jnp.concatenate(
            [v2[:, 1024 * q:1024 * (q + 1)] for q in range(4)], axis=0
        )
        o_ref[pl.ds(s * 1024, 1024), :] = jnp.transpose(t128, (1, 0))


def _tx(feat):
    return pl.pallas_call(
        _tx_body,
        grid=(B, RES // XS),
        in_specs=[
            pl.BlockSpec(
                (1, C, XS, RES, RES), lambda b, x: (b, 0, x, 0, 0)
            )
        ],
        out_specs=pl.BlockSpec(
            (XS * RES * RES // 4, 4 * C),
            lambda b, x: (b * (RES // XS) + x, 0),
        ),
        out_shape=jax.ShapeDtypeStruct((B * V // 4, 4 * C), jnp.float32),
    )(feat)


def _sc_body(
    table, idx_hbm, w_hbm, out_hbm,
    idx_all, w0, w1, w2, w3, w4, w5, rows_a, rows_b,
    out_a, out_b, semr, semo_a, semo_b,
):
    wid = lax.axis_index("s") * 2 + lax.axis_index("c")
    qbase0 = wid * PTS_PER_W
    wrefs = (w0, w1, w2, w3, w4, w5)

    pltpu.sync_copy(idx_hbm.at[:, pl.ds(qbase0, PTS_PER_W)], idx_all)
    for a in range(6):
        pltpu.sync_copy(w_hbm.at[a, pl.ds(qbase0, PTS_PER_W)], wrefs[a])

    def fire(g, rows):
        return [
            pltpu.async_copy(
                table.at[idx_all.at[k, pl.ds(g * CHUNK, CHUNK)]],
                rows.at[pl.ds(k * CHUNK, CHUNK)],
                semr,
            )
            for k in range(8)
        ]

    def compute(g, rows, outb):
        def p_body(p, carry_p):
            pv = jnp.full((LANES,), g * CHUNK, jnp.int32) + p
            wlx = plsc.load_gather(w0, [pv])
            wly = plsc.load_gather(w1, [pv])
            wlz = plsc.load_gather(w2, [pv])
            wrx = plsc.load_gather(w3, [pv])
            wry = plsc.load_gather(w4, [pv])
            wrz = plsc.load_gather(w5, [pv])
            for h in range(2):
                f = [
                    rows[k * CHUNK + p, pl.ds(h * LANES, LANES)]
                    for k in range(8)
                ]
                f00 = f[0] * wlx + f[1] * wrx
                f10 = f[2] * wlx + f[3] * wrx
                f01 = f[4] * wlx + f[5] * wrx
                f11 = f[6] * wlx + f[7] * wrx
                g0 = f00 * wly + f10 * wry
                g1 = f01 * wly + f11 * wry
                outb[p, pl.ds(h * LANES, LANES)] = g0 * wlz + g1 * wrz
            return carry_p

        lax.fori_loop(0, CHUNK, p_body, 0)

    def out_copy(g, outb, semo):
        return pltpu.make_async_copy(
            outb, out_hbm.at[pl.ds(qbase0 + g * CHUNK, CHUNK)], semo
        )

    row_cp = {0: fire(0, rows_a), 1: fire(1, rows_b)}
    out_cp = {}
    for g in range(NCHUNK):
        rows = rows_a if g % 2 == 0 else rows_b
        outb = out_a if g % 2 == 0 else out_b
        semo = semo_a if g % 2 == 0 else semo_b
        for cp in row_cp.pop(g):
            cp.wait()
        if g >= 2:
            out_cp.pop(g - 2).wait()
        compute(g, rows, outb)
        cp = out_copy(g, outb, semo)
        cp.start()
        out_cp[g] = cp
        if g + 2 < NCHUNK:
            row_cp[g + 2] = fire(g + 2, rows)
    out_cp.pop(NCHUNK - 2).wait()
    out_cp.pop(NCHUNK - 1).wait()


@functools.lru_cache(maxsize=1)
def _make_sc_gather():
    mesh = plsc.VectorSubcoreMesh(core_axis_name="c", subcore_axis_name="s")
    return pl.kernel(
        _sc_body,
        out_type=jax.ShapeDtypeStruct((B * N, C), jnp.float32),
        mesh=mesh,
        compiler_params=pltpu.CompilerParams(
            use_tc_tiling_on_sc=False, needs_layout_passes=False
        ),
        scratch_types=[
            pltpu.VMEM((8, PTS_PER_W), jnp.int32),
            pltpu.VMEM((PTS_PER_W,), jnp.float32),
            pltpu.VMEM((PTS_PER_W,), jnp.float32),
            pltpu.VMEM((PTS_PER_W,), jnp.float32),
            pltpu.VMEM((PTS_PER_W,), jnp.float32),
            pltpu.VMEM((PTS_PER_W,), jnp.float32),
            pltpu.VMEM((PTS_PER_W,), jnp.float32),
            pltpu.VMEM((8 * CHUNK, C), jnp.float32),
            pltpu.VMEM((8 * CHUNK, C), jnp.float32),
            pltpu.VMEM((CHUNK, C), jnp.float32),
            pltpu.VMEM((CHUNK, C), jnp.float32),
            pltpu.SemaphoreType.DMA,
            pltpu.SemaphoreType.DMA,
            pltpu.SemaphoreType.DMA,
        ],
    )


def kernel(pts, feat):
    pmin = jnp.min(pts, axis=2, keepdims=True)
    pn0 = pts - pmin
    mpe = jnp.max(jnp.sqrt(jnp.sum(pn0 * pn0, axis=1))) + EPS
    idx8, w8 = _prep(pts, mpe.reshape(1, 1))
    table = _tx(feat).reshape(B * V, C)
    out = _make_sc_gather()(table, idx8, w8)
    return out.reshape(B, N, C).transpose(0, 2, 1)


# speedup vs baseline: 1.5654x; 1.0425x over previous
import functools

import jax
import jax.numpy as jnp
from jax import lax
from jax.experimental import pallas as pl
from jax.experimental.pallas import tpu as pltpu
from jax.experimental.pallas import tpu_sc as plsc

RES = 64
V = RES ** 3
EPS = 1e-08
B = 4
C = 32
N = 16384

NW = 32
PTS_PER_W = (B * N) // NW
CHUNK = 128
NCHUNK = PTS_PER_W // CHUNK
LANES = 16


def _prep_body(pts_ref, mpe_ref, idx_ref, w_ref):
    pts = pts_ref[...]
    pmin = jnp.min(pts, axis=2, keepdims=True)
    pn = pts - pmin
    pn = pn / mpe_ref[0, 0]
    vox = pn * (RES - 1.0)
    il = jnp.floor(vox).astype(jnp.int32)
    ir = jnp.ceil(vox).astype(jnp.int32)
    wl = 1.0 - vox
    wr = vox
    boff = (jnp.arange(B, dtype=jnp.int32) * V)[:, None]
    for k in range(8):
        bx, by, bz = k & 1, (k >> 1) & 1, (k >> 2) & 1
        ix = (ir if bx else il)[:, 0]
        iy = (ir if by else il)[:, 1]
        iz = (ir if bz else il)[:, 2]
        m = iy * RES + iz
        idxk = (
            boff + ix * (RES * RES)
            + 4 * (m & (1024 - 1)) + (m >> 10)
        )
        for b in range(B):
            idx_ref[pl.ds(k, 1), pl.ds(b * N, N)] = idxk[b:b + 1]
    for a in range(3):
        for b in range(B):
            w_ref[pl.ds(a, 1), pl.ds(b * N, N)] = wl[b:b + 1, a]
            w_ref[pl.ds(3 + a, 1), pl.ds(b * N, N)] = wr[b:b + 1, a]


def _prep(pts, mpe):
    return pl.pallas_call(
        _prep_body,
        out_shape=[
            jax.ShapeDtypeStruct((8, B * N), jnp.int32),
            jax.ShapeDtypeStruct((6, B * N), jnp.float32),
        ],
    )(pts, mpe)


XS = 8


def _tx_body(f_ref, o_ref):
    for s in range(XS):
        val = f_ref[0, :, s]
        v2 = val.reshape(C, RES * RES)
        t128 = jnp.concatenate(
            [v2[:, 1024 * q:1024 * (q + 1)] for q in range(4)], axis=0
        )
        o_ref[pl.ds(s * 1024, 1024), :] = jnp.transpose(t128, (1, 0))


def _tx(feat):
    return pl.pallas_call(
        _tx_body,
        grid=(B, RES // XS),
        in_specs=[
            pl.BlockSpec(
                (1, C, XS, RES, RES), lambda b, x: (b, 0, x, 0, 0)
            )
        ],
        out_specs=pl.BlockSpec(
            (XS * RES * RES // 4, 4 * C),
            lambda b, x: (b * (RES // XS) + x, 0),
        ),
        out_shape=jax.ShapeDtypeStruct((B * V // 4, 4 * C), jnp.float32),
    )(feat)


def _sc_body(
    table, idx_hbm, w_hbm, out_hbm,
    idx_all, w0, w1, w2, w3, w4, w5, rows_a, rows_b,
    out_a, out_b, semr, semo_a, semo_b,
):
    wid = lax.axis_index("s") * 2 + lax.axis_index("c")
    qbase0 = wid * PTS_PER_W
    wrefs = (w0, w1, w2, w3, w4, w5)

    pltpu.sync_copy(idx_hbm.at[:, pl.ds(qbase0, PTS_PER_W)], idx_all)
    for a in range(6):
        pltpu.sync_copy(w_hbm.at[a, pl.ds(qbase0, PTS_PER_W)], wrefs[a])

    def fire(g, rows):
        return [
            pltpu.async_copy(
                table.at[idx_all.at[k, pl.ds(g * CHUNK, CHUNK)]],
                rows.at[pl.ds(k * CHUNK, CHUNK)],
                semr,
            )
            for k in range(8)
        ]

    def compute(g, rows, outb):
        def p_body(p, carry_p):
            pv = jnp.full((LANES,), g * CHUNK, jnp.int32) + p
            wlx = plsc.load_gather(w0, [pv])
            wly = plsc.load_gather(w1, [pv])
            wlz = plsc.load_gather(w2, [pv])
            wrx = plsc.load_gather(w3, [pv])
            wry = plsc.load_gather(w4, [pv])
            wrz = plsc.load_gather(w5, [pv])
            for h in range(2):
                f = [
                    rows[k * CHUNK + p, pl.ds(h * LANES, LANES)]
                    for k in range(8)
                ]
                f00 = f[0] * wlx + f[1] * wrx
                f10 = f[2] * wlx + f[3] * wrx
                f01 = f[4] * wlx + f[5] * wrx
                f11 = f[6] * wlx + f[7] * wrx
                g0 = f00 * wly + f10 * wry
                g1 = f01 * wly + f11 * wry
                outb[p, pl.ds(h * LANES, LANES)] = g0 * wlz + g1 * wrz
            return carry_p

        lax.fori_loop(0, CHUNK, p_body, 0)

    def out_copy(g, outb, semo):
        return pltpu.make_async_copy(
            outb, out_hbm.at[pl.ds(qbase0 + g * CHUNK, CHUNK)], semo
        )

    row_cp = {0: fire(0, rows_a), 1: fire(1, rows_b)}
    out_cp = {}
    for g in range(NCHUNK):
        rows = rows_a if g % 2 == 0 else rows_b
        outb = out_a if g % 2 == 0 else out_b
        semo = semo_a if g % 2 == 0 else semo_b
        for cp in row_cp.pop(g):
            cp.wait()
        if g >= 2:
            out_cp.pop(g - 2).wait()
        compute(g, rows, outb)
        cp = out_copy(g, outb, semo)
        cp.start()
        out_cp[g] = cp
        if g + 2 < NCHUNK:
            row_cp[g + 2] = fire(g + 2, rows)
    out_cp.pop(NCHUNK - 2).wait()
    out_cp.pop(NCHUNK - 1).wait()


@functools.lru_cache(maxsize=1)
def _make_sc_gather():
    mesh = plsc.VectorSubcoreMesh(core_axis_name="c", subcore_axis_name="s")
    return pl.kernel(
        _sc_body,
        out_type=jax.ShapeDtypeStruct((B * N, C), jnp.float32),
        mesh=mesh,
        compiler_params=pltpu.CompilerParams(
            use_tc_tiling_on_sc=False, needs_layout_passes=False
        ),
        scratch_types=[
            pltpu.VMEM((8, PTS_PER_W), jnp.int32),
            pltpu.VMEM((PTS_PER_W,), jnp.float32),
            pltpu.VMEM((PTS_PER_W,), jnp.float32),
            pltpu.VMEM((PTS_PER_W,), jnp.float32),
            pltpu.VMEM((PTS_PER_W,), jnp.float32),
            pltpu.VMEM((PTS_PER_W,), jnp.float32),
            pltpu.VMEM((PTS_PER_W,), jnp.float32),
            pltpu.VMEM((8 * CHUNK, C), jnp.float32),
            pltpu.VMEM((8 * CHUNK, C), jnp.float32),
            pltpu.VMEM((CHUNK, C), jnp.float32),
            pltpu.VMEM((CHUNK, C), jnp.float32),
            pltpu.SemaphoreType.DMA,
            pltpu.SemaphoreType.DMA,
            pltpu.SemaphoreType.DMA,
        ],
    )


def kernel(pts, feat):
    pmin = jnp.min(pts, axis=2, keepdims=True)
    pn0 = pts - pmin
    mpe = jnp.max(jnp.sqrt(jnp.sum(pn0 * pn0, axis=1))) + EPS
    idx8, w8 = _prep(pts, mpe.reshape(1, 1))
    table = _tx(feat).reshape(B * V, C)
    out = _make_sc_gather()(table, idx8, w8)
    return out.reshape(B, N, C).transpose(0, 2, 1)
